# baseline (speedup 1.0000x reference)
import jax
import jax.numpy as jnp
from jax import lax
from jax.experimental import pallas as pl
from jax.experimental.pallas import tpu as pltpu
from jax.experimental.pallas import tpu_sc as plsc

BATCH = 4096
HIST = 200
DIM = 64
N = BATCH * HIST
NC, NS = 2, 16
NW = NC * NS
PER_W = N // NW
U = 16
MC = U * 16
G = PER_W // MC


def _body(idx_hbm, table_hbm, out_hbm, idx_v, rows_v, sem_g, sem_s):
    wid = lax.axis_index("s") * NC + lax.axis_index("c")
    pltpu.sync_copy(idx_hbm.at[wid], idx_v)
    base = wid * PER_W

    def fire(m, buf):
        for u in range(U):
            ivec = idx_v[pl.ds(m * MC + u * 16, 16)]
            pltpu.async_copy(table_hbm.at[ivec], rows_v.at[buf, pl.ds(u * 16, 16)], sem_g)

    def drain(buf):
        for u in range(U):
            pltpu.make_async_copy(
                table_hbm.at[idx_v[pl.ds(0, 16)]],
                rows_v.at[buf, pl.ds(u * 16, 16)], sem_g).wait()

    def store_start(m, buf):
        pltpu.async_copy(rows_v.at[buf], out_hbm.at[pl.ds(base + m * MC, MC)], sem_s)

    def store_wait(buf):
        pltpu.make_async_copy(rows_v.at[buf], out_hbm.at[pl.ds(base, MC)], sem_s).wait()

    fire(0, 0)

    def outer(o, carry):
        for half in range(2):
            m = o * 2 + half

            @pl.when(m + 1 < G)
            def _():
                @pl.when(m >= 1)
                def _():
                    store_wait(1 - half)
                fire(m + 1, 1 - half)

            drain(half)
            store_start(m, half)
        return carry

    lax.fori_loop(0, G // 2, outer, 0)
    store_wait(0)
    store_wait(1)


@jax.jit
def _gather(idx2, weight):
    mesh = plsc.VectorSubcoreMesh(core_axis_name="c", subcore_axis_name="s")
    return pl.kernel(
        _body,
        out_type=jax.ShapeDtypeStruct((N, DIM), jnp.float32),
        mesh=mesh,
        scratch_types=[
            pltpu.VMEM((PER_W,), jnp.int32),
            pltpu.VMEM((2, MC, DIM), jnp.float32),
            pltpu.SemaphoreType.DMA,
            pltpu.SemaphoreType.DMA,
        ],
        compiler_params=pltpu.CompilerParams(use_tc_tiling_on_sc=False),
    )(idx2, weight)


def kernel(input_, weight):
    idx2 = input_.astype(jnp.int32).reshape(NW, PER_W)
    out = _gather(idx2, weight)
    return out.reshape(BATCH, HIST, DIM)
